# baseline (speedup 1.0000x reference)
import functools

import jax
import jax.numpy as jnp
from jax import lax
from jax.experimental import pallas as pl
from jax.experimental.pallas import tpu as pltpu
from jax.experimental.pallas import tpu_sc as plsc

_N = 10000
_E = 320000
_H = 128
_C = 64
_NP = 10240
_EP = 327680
_EPT = _EP // 32
_BE = 64
_NBLK = _EPT // _BE
_ROWS_PT = _NP // 16
_NZC = _ROWS_PT // _BE
_NEG = 0.2


@functools.cache
def _make_sc_layer(D):
    mesh = plsc.VectorSubcoreMesh(core_axis_name="c", subcore_axis_name="s")

    @functools.partial(
        pl.kernel,
        out_type=[
            jax.ShapeDtypeStruct((2, _NP, D), jnp.float32),
            jax.ShapeDtypeStruct((2, _NP), jnp.float32),
        ],
        mesh=mesh,
        scratch_types=[
            pltpu.VMEM((_N,), jnp.float32),
            pltpu.VMEM((_N,), jnp.float32),
            pltpu.VMEM((2, _BE, D), jnp.float32),
            pltpu.VMEM((4, _BE), jnp.int32),
            pltpu.VMEM((4, _BE), jnp.int32),
            pltpu.VMEM((2, _BE), jnp.float32),
            pltpu.VMEM((_BE,), jnp.float32),
            pltpu.VMEM((16,), jnp.float32),
            pltpu.VMEM_SHARED((_NP, D), jnp.float32),
            pltpu.VMEM_SHARED((_NP,), jnp.float32),
            [pltpu.SemaphoreType.DMA] * 2,
            [pltpu.SemaphoreType.DMA] * 2,
            [pltpu.SemaphoreType.DMA] * 2,
            [pltpu.SemaphoreType.DMA] * 4,
        ],
        compiler_params=pltpu.CompilerParams(needs_layout_passes=False),
    )
    def sc_fn(feat_h, el_h, er_h, src_h, dst_h, mv_h, p_h, d_h,
              el_v, er_v, rows_v, srcblk_v, dstblk_v, eeblk_v,
              zrow_v, mvec_v, sh_p, sh_d, gsem, spsem, sdsem, stsem):
        c = lax.axis_index("c")
        s = lax.axis_index("s")
        wid = c * 16 + s
        ebase = wid * _EPT

        pltpu.sync_copy(el_h, el_v)
        pltpu.sync_copy(er_h, er_v)
        pltpu.sync_copy(mv_h, mvec_v)
        Mv = mvec_v[...]

        iota16 = lax.broadcasted_iota(jnp.int32, (16,), 0)
        zero16 = jnp.zeros((16,), jnp.float32)

        def stage_start(b, slot):
            eb = ebase + b * _BE
            pltpu.async_copy(src_h.at[pl.ds(eb, _BE)], srcblk_v.at[slot],
                             stsem[slot])
            pltpu.async_copy(dst_h.at[pl.ds(eb, _BE)], dstblk_v.at[slot],
                             stsem[slot])

        def stage_wait(b, slot):
            eb = ebase + b * _BE
            pltpu.make_async_copy(src_h.at[pl.ds(eb, _BE)], srcblk_v.at[slot],
                                  stsem[slot]).wait()
            pltpu.make_async_copy(dst_h.at[pl.ds(eb, _BE)], dstblk_v.at[slot],
                                  stsem[slot]).wait()

        def gather_start(slot, q):
            pltpu.async_copy(feat_h.at[srcblk_v.at[slot]], rows_v.at[q],
                             gsem[q])

        def gather_wait(slot, q):
            pltpu.make_async_copy(feat_h.at[srcblk_v.at[slot]], rows_v.at[q],
                                  gsem[q]).wait()

        def scatter_start(slot, q):
            pltpu.async_copy(eeblk_v.at[q], sh_d.at[dstblk_v.at[slot]],
                             sdsem[q], add=True)
            pltpu.async_copy(rows_v.at[q], sh_p.at[dstblk_v.at[slot]],
                             spsem[q], add=True)

        def scatter_wait(slot, q):
            pltpu.make_async_copy(eeblk_v.at[q], sh_d.at[dstblk_v.at[slot]],
                                  sdsem[q]).wait()
            pltpu.make_async_copy(rows_v.at[q], sh_p.at[dstblk_v.at[slot]],
                                  spsem[q]).wait()

        def compute_block(b, slot, q):
            eb = ebase + b * _BE
            for j in range(_BE // 16):
                sl = pl.ds(j * 16, 16)
                e = (plsc.load_gather(el_v, [srcblk_v[slot, sl]])
                     + plsc.load_gather(er_v, [dstblk_v[slot, sl]]))
                e = jnp.where(e >= 0.0, e, _NEG * e)
                gid = eb + j * 16 + iota16
                e = jnp.where(gid < _E, e, -1e30)
                eeblk_v[q, sl] = jnp.exp(e - Mv)

            def body_r(r, u):
                av = plsc.load_gather(eeblk_v.at[q],
                                      [jnp.full((16,), r, jnp.int32)])
                for j in range(D // 16):
                    sl = pl.ds(j * 16, 16)
                    rows_v[q, r, sl] = rows_v[q, r, sl] * av
                return u

            lax.fori_loop(0, _BE, body_r, 0)

        def body_z(r, t):
            for j in range(D // 16):
                rows_v[0, r, pl.ds(j * 16, 16)] = zero16
            return t

        lax.fori_loop(0, _BE, body_z, 0)
        for j in range(_BE // 16):
            zrow_v[pl.ds(j * 16, 16)] = zero16
        for k in range(_NZC):
            base = s * _ROWS_PT + k * _BE
            pltpu.sync_copy(rows_v.at[0], sh_p.at[pl.ds(base, _BE)])
            pltpu.sync_copy(zrow_v, sh_d.at[pl.ds(base, _BE)])

        plsc.subcore_barrier()

        stage_start(0, 0)
        stage_start(1, 1)
        stage_start(2, 2)
        stage_wait(0, 0)
        gather_start(0, 0)

        def body_g(g, t):
            for qq in range(4):
                b = g * 4 + qq
                slot = qq
                q = qq % 2

                gather_wait(slot, q)
                compute_block(b, slot, q)

                @pl.when(b >= 1)
                def _():
                    scatter_wait((slot - 1) % 4, (q + 1) % 2)

                scatter_start(slot, q)

                @pl.when(b + 1 < _NBLK)
                def _():
                    stage_wait(b + 1, (slot + 1) % 4)
                    gather_start((slot + 1) % 4, (q + 1) % 2)

                @pl.when(b + 3 < _NBLK)
                def _():
                    stage_start(b + 3, (slot + 3) % 4)
            return t

        lax.fori_loop(0, _NBLK // 4, body_g, 0)
        scatter_wait(3, 1)

        plsc.subcore_barrier()

        for k in range(_NZC):
            base = s * _ROWS_PT + k * _BE
            pltpu.sync_copy(sh_p.at[pl.ds(base, _BE)],
                            p_h.at[c].at[pl.ds(base, _BE)])

        @pl.when(s == 0)
        def _():
            pltpu.sync_copy(sh_d, d_h.at[c])

    return sc_fn


def _tc_pre_body(x_ref, w_ref, alr_ref, feat_ref, eler_ref, emax_ref):
    feat = jnp.dot(x_ref[...], w_ref[...], preferred_element_type=jnp.float32)
    feat_ref[...] = feat
    eler = jnp.dot(feat, alr_ref[...], preferred_element_type=jnp.float32)
    eler_ref[...] = eler
    emax_ref[...] = jnp.max(eler, axis=0, keepdims=True)


_tc_pre = pl.pallas_call(
    _tc_pre_body,
    out_shape=[
        jax.ShapeDtypeStruct((_N, _H), jnp.float32),
        jax.ShapeDtypeStruct((_N, 8), jnp.float32),
        jax.ShapeDtypeStruct((1, 8), jnp.float32),
    ],
)


def _make_tc_mid(has_res, Dn):
    def body(*refs):
        if has_res:
            (p0, p1, d0, d1, hprev, b, wn, alrn,
             h_ref, feat_ref, eler_ref, emax_ref) = refs
        else:
            (p0, p1, d0, d1, b, wn, alrn,
             h_ref, feat_ref, eler_ref, emax_ref) = refs
        num = p0[...] + p1[...]
        den = d0[...] + d1[...]
        agg = jnp.where(den > 0.0, num / den, 0.0)
        h = agg + b[...]
        if has_res:
            h = h + hprev[...]
        h = jnp.where(h > 0.0, h, jnp.exp(h) - 1.0)
        h_ref[...] = h
        feat = jnp.dot(h, wn[...], preferred_element_type=jnp.float32)
        feat_ref[...] = feat
        eler = jnp.dot(feat, alrn[...], preferred_element_type=jnp.float32)
        eler_ref[...] = eler
        emax_ref[...] = jnp.max(eler, axis=0, keepdims=True)

    return pl.pallas_call(
        body,
        out_shape=[
            jax.ShapeDtypeStruct((_N, _H), jnp.float32),
            jax.ShapeDtypeStruct((_N, Dn), jnp.float32),
            jax.ShapeDtypeStruct((_N, 8), jnp.float32),
            jax.ShapeDtypeStruct((1, 8), jnp.float32),
        ],
    )


_tc_mid0 = _make_tc_mid(False, _H)
_tc_mid1 = _make_tc_mid(True, _H)


def _tc_fin_body(p0, p1, d0, d1, hprev, wres, b, out_ref):
    num = p0[...] + p1[...]
    den = d0[...] + d1[...]
    agg = jnp.where(den > 0.0, num / den, 0.0)
    res = jnp.dot(hprev[...], wres[...], preferred_element_type=jnp.float32)
    out_ref[...] = agg + res + b[...]


_tc_fin = pl.pallas_call(
    _tc_fin_body,
    out_shape=jax.ShapeDtypeStruct((_N, _C), jnp.float32),
)


def _alr(al, ar):
    z = jnp.zeros_like(al)
    return jnp.stack([al, ar, z, z, z, z, z, z], axis=1)


def kernel(inputs, edge_index, W0, al0, ar0, b0, W1, al1, ar1, b1,
           W2, al2, ar2, b2, Wres2):
    src = edge_index[0]
    dst = edge_index[1]
    pad = jnp.zeros((_EP - _E,), jnp.int32)
    srcp = jnp.concatenate([src, pad])
    dstp = jnp.concatenate([dst, pad])

    def shift_vec(emax):
        m0 = emax[0, 0] + emax[0, 1]
        m = jnp.where(m0 >= 0.0, m0, _NEG * m0)
        return jnp.full((16,), m, jnp.float32)

    _sc_layer_h = _make_sc_layer(_H)
    zpad = jnp.zeros((_H, _H - _C), jnp.float32)
    W2p = jnp.concatenate([W2, zpad], axis=1)
    al2p = jnp.concatenate([al2, jnp.zeros((_H - _C,), jnp.float32)])
    ar2p = jnp.concatenate([ar2, jnp.zeros((_H - _C,), jnp.float32)])

    feat0, eler0, emax0 = _tc_pre(inputs, W0, _alr(al0, ar0))
    p, d = _sc_layer_h(feat0, eler0[:, 0], eler0[:, 1], srcp, dstp,
                       shift_vec(emax0))
    h1, feat1, eler1, emax1 = _tc_mid0(p[0, :_N], p[1, :_N], d[0, :_N, None],
                                       d[1, :_N, None], b0, W1,
                                       _alr(al1, ar1))

    p, d = _sc_layer_h(feat1, eler1[:, 0], eler1[:, 1], srcp, dstp,
                       shift_vec(emax1))
    h2, feat2, eler2, emax2 = _tc_mid1(p[0, :_N], p[1, :_N], d[0, :_N, None],
                                       d[1, :_N, None], h1, b1, W2p,
                                       _alr(al2p, ar2p))

    p, d = _sc_layer_h(feat2, eler2[:, 0], eler2[:, 1], srcp, dstp,
                       shift_vec(emax2))
    logits = _tc_fin(p[0, :_N, :_C], p[1, :_N, :_C], d[0, :_N, None],
                     d[1, :_N, None], h2, Wres2, b2)
    return logits


# speedup vs baseline: 1.4463x; 1.4463x over previous
import functools

import jax
import jax.numpy as jnp
from jax import lax
from jax.experimental import pallas as pl
from jax.experimental.pallas import tpu as pltpu
from jax.experimental.pallas import tpu_sc as plsc

_N = 10000
_E = 320000
_H = 128
_C = 64
_NP = 10240
_EP = 327680
_EPT = _EP // 32
_BE = 128
_NBLK = _EPT // _BE
_ROWS_PT = _NP // 16
_NZC = _ROWS_PT // _BE
_NEG = 0.2


@functools.cache
def _make_sc_layer(D):
    mesh = plsc.VectorSubcoreMesh(core_axis_name="c", subcore_axis_name="s")

    @functools.partial(
        pl.kernel,
        out_type=[
            jax.ShapeDtypeStruct((2, _NP, D), jnp.float32),
            jax.ShapeDtypeStruct((2, _NP), jnp.float32),
        ],
        mesh=mesh,
        scratch_types=[
            pltpu.VMEM((2, _BE, D), jnp.float32),
            pltpu.VMEM((4, 2, _BE), jnp.int32),
            pltpu.VMEM((2, _BE), jnp.float32),
            pltpu.VMEM((2, _BE), jnp.float32),
            pltpu.VMEM((2, _BE), jnp.float32),
            pltpu.VMEM((_BE,), jnp.float32),
            pltpu.VMEM((16,), jnp.float32),
            pltpu.VMEM_SHARED((_NP, D), jnp.float32),
            pltpu.VMEM_SHARED((_NP,), jnp.float32),
            pltpu.VMEM_SHARED((_N,), jnp.float32),
            pltpu.VMEM_SHARED((_N,), jnp.float32),
            [pltpu.SemaphoreType.DMA] * 2,
            [pltpu.SemaphoreType.DMA] * 2,
            [pltpu.SemaphoreType.DMA] * 2,
            [pltpu.SemaphoreType.DMA] * 2,
            [pltpu.SemaphoreType.DMA] * 2,
            [pltpu.SemaphoreType.DMA] * 4,
        ],
        compiler_params=pltpu.CompilerParams(needs_layout_passes=False),
    )
    def sc_fn(feat_h, sd_h, mv_h, el_h, er_h, p_h, d_h,
              rows_v, sdblk_v, elblk_v, erblk_v, eeblk_v, zrow_v, mvec_v,
              sh_p, sh_d, sh_el, sh_er,
              gsem, elsem, ersem, spsem, sdsem, stsem):
        c = lax.axis_index("c")
        s = lax.axis_index("s")
        wid = c * 16 + s
        ebase = wid * _EPT

        pltpu.sync_copy(mv_h, mvec_v)
        Mv = mvec_v[...]

        @pl.when(s == 0)
        def _():
            pltpu.sync_copy(el_h, sh_el)
            pltpu.sync_copy(er_h, sh_er)

        iota16 = lax.broadcasted_iota(jnp.int32, (16,), 0)
        zero16 = jnp.zeros((16,), jnp.float32)

        def stage_start(b, slot):
            eb = ebase + b * _BE
            pltpu.async_copy(sd_h.at[:, pl.ds(eb, _BE)], sdblk_v.at[slot],
                             stsem[slot])

        def stage_wait(b, slot):
            eb = ebase + b * _BE
            pltpu.make_async_copy(sd_h.at[:, pl.ds(eb, _BE)],
                                  sdblk_v.at[slot], stsem[slot]).wait()

        def gathers_start(slot, q):
            pltpu.async_copy(sh_el.at[sdblk_v.at[slot, 0]], elblk_v.at[q],
                             elsem[q])
            pltpu.async_copy(sh_er.at[sdblk_v.at[slot, 1]], erblk_v.at[q],
                             ersem[q])
            pltpu.async_copy(feat_h.at[sdblk_v.at[slot, 0]], rows_v.at[q],
                             gsem[q])

        def gathers_wait(slot, q):
            pltpu.make_async_copy(sh_el.at[sdblk_v.at[slot, 0]],
                                  elblk_v.at[q], elsem[q]).wait()
            pltpu.make_async_copy(sh_er.at[sdblk_v.at[slot, 1]],
                                  erblk_v.at[q], ersem[q]).wait()
            pltpu.make_async_copy(feat_h.at[sdblk_v.at[slot, 0]],
                                  rows_v.at[q], gsem[q]).wait()

        def scatter_start(slot, q):
            pltpu.async_copy(eeblk_v.at[q], sh_d.at[sdblk_v.at[slot, 1]],
                             sdsem[q], add=True)
            pltpu.async_copy(rows_v.at[q], sh_p.at[sdblk_v.at[slot, 1]],
                             spsem[q], add=True)

        def scatter_wait(slot, q):
            pltpu.make_async_copy(eeblk_v.at[q], sh_d.at[sdblk_v.at[slot, 1]],
                                  sdsem[q]).wait()
            pltpu.make_async_copy(rows_v.at[q], sh_p.at[sdblk_v.at[slot, 1]],
                                  spsem[q]).wait()

        def compute_block(b, slot, q):
            eb = ebase + b * _BE
            for j in range(_BE // 16):
                sl = pl.ds(j * 16, 16)
                e = elblk_v[q, sl] + erblk_v[q, sl]
                e = jnp.where(e >= 0.0, e, _NEG * e)
                gid = eb + j * 16 + iota16
                e = jnp.where(gid < _E, e, -1e30)
                eeblk_v[q, sl] = jnp.exp(e - Mv)

            def body_r(i, u):
                for rr in range(2):
                    r = i * 2 + rr
                    av = plsc.load_gather(eeblk_v.at[q],
                                          [jnp.full((16,), r, jnp.int32)])
                    for j in range(D // 16):
                        sl = pl.ds(j * 16, 16)
                        rows_v[q, r, sl] = rows_v[q, r, sl] * av
                return u

            lax.fori_loop(0, _BE // 2, body_r, 0)

        def body_z(r, t):
            for j in range(D // 16):
                rows_v[0, r, pl.ds(j * 16, 16)] = zero16
            return t

        lax.fori_loop(0, _BE, body_z, 0)
        for j in range(_BE // 16):
            zrow_v[pl.ds(j * 16, 16)] = zero16
        for k in range(_NZC):
            base = s * _ROWS_PT + k * _BE
            pltpu.sync_copy(rows_v.at[0], sh_p.at[pl.ds(base, _BE)])
            pltpu.sync_copy(zrow_v, sh_d.at[pl.ds(base, _BE)])

        plsc.subcore_barrier()

        stage_start(0, 0)
        stage_start(1, 1)
        stage_start(2, 2)
        stage_wait(0, 0)
        gathers_start(0, 0)

        def body_g(g, t):
            for qq in range(4):
                b = g * 4 + qq
                slot = qq
                q = qq % 2

                @pl.when(b >= 1)
                def _():
                    scatter_wait((slot - 1) % 4, (q + 1) % 2)

                @pl.when(b + 1 < _NBLK)
                def _():
                    stage_wait(b + 1, (slot + 1) % 4)
                    gathers_start((slot + 1) % 4, (q + 1) % 2)

                @pl.when(b + 3 < _NBLK)
                def _():
                    stage_start(b + 3, (slot + 3) % 4)

                gathers_wait(slot, q)
                compute_block(b, slot, q)
                scatter_start(slot, q)
            return t

        lax.fori_loop(0, _NBLK // 4, body_g, 0)
        scatter_wait(3, 1)

        plsc.subcore_barrier()

        for k in range(_NZC):
            base = s * _ROWS_PT + k * _BE
            pltpu.sync_copy(sh_p.at[pl.ds(base, _BE)],
                            p_h.at[c].at[pl.ds(base, _BE)])

        @pl.when(s == 0)
        def _():
            pltpu.sync_copy(sh_d, d_h.at[c])

    return sc_fn


def _tc_pre_body(x_ref, w_ref, alr_ref, feat_ref, eler_ref, emax_ref):
    feat = jnp.dot(x_ref[...], w_ref[...], preferred_element_type=jnp.float32)
    feat_ref[...] = feat
    eler = jnp.dot(feat, alr_ref[...], preferred_element_type=jnp.float32)
    eler_ref[...] = eler
    emax_ref[...] = jnp.max(eler, axis=0, keepdims=True)


_tc_pre = pl.pallas_call(
    _tc_pre_body,
    out_shape=[
        jax.ShapeDtypeStruct((_N, _H), jnp.float32),
        jax.ShapeDtypeStruct((_N, 8), jnp.float32),
        jax.ShapeDtypeStruct((1, 8), jnp.float32),
    ],
)


def _make_tc_mid(has_res, Dn):
    def body(*refs):
        if has_res:
            (p0, p1, d0, d1, hprev, b, wn, alrn,
             h_ref, feat_ref, eler_ref, emax_ref) = refs
        else:
            (p0, p1, d0, d1, b, wn, alrn,
             h_ref, feat_ref, eler_ref, emax_ref) = refs
        num = p0[...] + p1[...]
        den = d0[...] + d1[...]
        agg = jnp.where(den > 0.0, num / den, 0.0)
        h = agg + b[...]
        if has_res:
            h = h + hprev[...]
        h = jnp.where(h > 0.0, h, jnp.exp(h) - 1.0)
        h_ref[...] = h
        feat = jnp.dot(h, wn[...], preferred_element_type=jnp.float32)
        feat_ref[...] = feat
        eler = jnp.dot(feat, alrn[...], preferred_element_type=jnp.float32)
        eler_ref[...] = eler
        emax_ref[...] = jnp.max(eler, axis=0, keepdims=True)

    return pl.pallas_call(
        body,
        out_shape=[
            jax.ShapeDtypeStruct((_N, _H), jnp.float32),
            jax.ShapeDtypeStruct((_N, Dn), jnp.float32),
            jax.ShapeDtypeStruct((_N, 8), jnp.float32),
            jax.ShapeDtypeStruct((1, 8), jnp.float32),
        ],
    )


_tc_mid0 = _make_tc_mid(False, _H)
_tc_mid1 = _make_tc_mid(True, _H)


def _tc_fin_body(p0, p1, d0, d1, hprev, wres, b, out_ref):
    num = p0[...] + p1[...]
    den = d0[...] + d1[...]
    agg = jnp.where(den > 0.0, num / den, 0.0)
    res = jnp.dot(hprev[...], wres[...], preferred_element_type=jnp.float32)
    out_ref[...] = agg + res + b[...]


_tc_fin = pl.pallas_call(
    _tc_fin_body,
    out_shape=jax.ShapeDtypeStruct((_N, _C), jnp.float32),
)


def _alr(al, ar):
    z = jnp.zeros_like(al)
    return jnp.stack([al, ar, z, z, z, z, z, z], axis=1)


def kernel(inputs, edge_index, W0, al0, ar0, b0, W1, al1, ar1, b1,
           W2, al2, ar2, b2, Wres2):
    sdp = jnp.pad(edge_index, ((0, 0), (0, _EP - _E)))

    def shift_vec(emax):
        m0 = emax[0, 0] + emax[0, 1]
        m = jnp.where(m0 >= 0.0, m0, _NEG * m0)
        return jnp.full((16,), m, jnp.float32)

    _sc_layer_h = _make_sc_layer(_H)
    zpad = jnp.zeros((_H, _H - _C), jnp.float32)
    W2p = jnp.concatenate([W2, zpad], axis=1)
    al2p = jnp.concatenate([al2, jnp.zeros((_H - _C,), jnp.float32)])
    ar2p = jnp.concatenate([ar2, jnp.zeros((_H - _C,), jnp.float32)])

    feat0, eler0, emax0 = _tc_pre(inputs, W0, _alr(al0, ar0))
    p, d = _sc_layer_h(feat0, sdp, shift_vec(emax0),
                       eler0[:, 0], eler0[:, 1])
    h1, feat1, eler1, emax1 = _tc_mid0(p[0, :_N], p[1, :_N], d[0, :_N, None],
                                       d[1, :_N, None], b0, W1,
                                       _alr(al1, ar1))

    p, d = _sc_layer_h(feat1, sdp, shift_vec(emax1),
                       eler1[:, 0], eler1[:, 1])
    h2, feat2, eler2, emax2 = _tc_mid1(p[0, :_N], p[1, :_N], d[0, :_N, None],
                                       d[1, :_N, None], h1, b1, W2p,
                                       _alr(al2p, ar2p))

    p, d = _sc_layer_h(feat2, sdp, shift_vec(emax2),
                       eler2[:, 0], eler2[:, 1])
    logits = _tc_fin(p[0, :_N, :_C], p[1, :_N, :_C], d[0, :_N, None],
                     d[1, :_N, None], h2, Wres2, b2)
    return logits


# speedup vs baseline: 1.5355x; 1.0617x over previous
import functools

import jax
import jax.numpy as jnp
from jax import lax
from jax.experimental import pallas as pl
from jax.experimental.pallas import tpu as pltpu
from jax.experimental.pallas import tpu_sc as plsc

_N = 10000
_E = 320000
_H = 128
_C = 64
_NP = 10240
_EP = 327680
_BE = 128
_EPT0 = 14336
_EPT1 = 6144
_NBLK0 = _EPT0 // _BE
_NBLK1 = _EPT1 // _BE
_ROWS_PT = _NP // 16
_NZC = _ROWS_PT // _BE
_NEG = 0.2


@functools.cache
def _make_sc_layer(D):
    mesh = plsc.VectorSubcoreMesh(core_axis_name="c", subcore_axis_name="s")

    @functools.partial(
        pl.kernel,
        out_type=[
            jax.ShapeDtypeStruct((2, _NP, D), jnp.float32),
            jax.ShapeDtypeStruct((2, _NP), jnp.float32),
        ],
        mesh=mesh,
        scratch_types=[
            pltpu.VMEM((2, _BE, D), jnp.float32),
            pltpu.VMEM((4, 2, _BE), jnp.int32),
            pltpu.VMEM((2, _BE), jnp.float32),
            pltpu.VMEM((2, _BE), jnp.float32),
            pltpu.VMEM((2, _BE), jnp.float32),
            pltpu.VMEM((_BE,), jnp.float32),
            pltpu.VMEM((16,), jnp.float32),
            pltpu.VMEM_SHARED((_NP, D), jnp.float32),
            pltpu.VMEM_SHARED((_NP,), jnp.float32),
            pltpu.VMEM_SHARED((_N,), jnp.float32),
            pltpu.VMEM_SHARED((_N,), jnp.float32),
            [pltpu.SemaphoreType.DMA] * 2,
            [pltpu.SemaphoreType.DMA] * 2,
            [pltpu.SemaphoreType.DMA] * 2,
            [pltpu.SemaphoreType.DMA] * 2,
            [pltpu.SemaphoreType.DMA] * 2,
            [pltpu.SemaphoreType.DMA] * 4,
        ],
        compiler_params=pltpu.CompilerParams(needs_layout_passes=False),
    )
    def sc_fn(feat_h, sd_h, mv_h, el_h, er_h, p_h, d_h,
              rows_v, sdblk_v, elblk_v, erblk_v, eeblk_v,
              zrow_v, mvec_v, sh_p, sh_d, sh_el, sh_er,
              gsem, elsem, ersem, spsem, sdsem, stsem):
        c = lax.axis_index("c")
        s = lax.axis_index("s")
        ebase = jnp.where(c == 0, s * _EPT0, 16 * _EPT0 + s * _EPT1)
        nblk = jnp.where(c == 0, _NBLK0, _NBLK1)

        pltpu.sync_copy(mv_h, mvec_v)
        Mv = mvec_v[...]

        @pl.when(s == 0)
        def _():
            pltpu.sync_copy(el_h, sh_el)
            pltpu.sync_copy(er_h, sh_er)

        iota16 = lax.broadcasted_iota(jnp.int32, (16,), 0)
        zero16 = jnp.zeros((16,), jnp.float32)

        def stage_start(b, slot):
            eb = ebase + b * _BE
            pltpu.async_copy(sd_h.at[:, pl.ds(eb, _BE)], sdblk_v.at[slot],
                             stsem[slot])

        def stage_wait(b, slot):
            eb = ebase + b * _BE
            pltpu.make_async_copy(sd_h.at[:, pl.ds(eb, _BE)],
                                  sdblk_v.at[slot], stsem[slot]).wait()

        def gathers_start(slot, q):
            pltpu.async_copy(sh_el.at[sdblk_v.at[slot, 0]], elblk_v.at[q],
                             elsem[q])
            pltpu.async_copy(sh_er.at[sdblk_v.at[slot, 1]], erblk_v.at[q],
                             ersem[q])
            pltpu.async_copy(feat_h.at[sdblk_v.at[slot, 0]], rows_v.at[q],
                             gsem[q])

        def gathers_wait(slot, q):
            pltpu.make_async_copy(sh_el.at[sdblk_v.at[slot, 0]],
                                  elblk_v.at[q], elsem[q]).wait()
            pltpu.make_async_copy(sh_er.at[sdblk_v.at[slot, 1]],
                                  erblk_v.at[q], ersem[q]).wait()
            pltpu.make_async_copy(feat_h.at[sdblk_v.at[slot, 0]],
                                  rows_v.at[q], gsem[q]).wait()

        def scatter_start(slot, q):
            pltpu.async_copy(eeblk_v.at[q], sh_d.at[sdblk_v.at[slot, 1]],
                             sdsem[q], add=True)
            pltpu.async_copy(rows_v.at[q], sh_p.at[sdblk_v.at[slot, 1]],
                             spsem[q], add=True)

        def scatter_wait(slot, q):
            pltpu.make_async_copy(eeblk_v.at[q], sh_d.at[sdblk_v.at[slot, 1]],
                                  sdsem[q]).wait()
            pltpu.make_async_copy(rows_v.at[q], sh_p.at[sdblk_v.at[slot, 1]],
                                  spsem[q]).wait()

        def compute_block(b, slot, q):
            eb = ebase + b * _BE
            for j in range(_BE // 16):
                sl = pl.ds(j * 16, 16)
                e = elblk_v[q, sl] + erblk_v[q, sl]
                e = jnp.where(e >= 0.0, e, _NEG * e)
                gid = eb + j * 16 + iota16
                e = jnp.where(gid < _E, e, -1e30)
                eeblk_v[q, sl] = jnp.exp(e - Mv)

            def body_r(i, u):
                for rr in range(2):
                    r = i * 2 + rr
                    av = plsc.load_gather(eeblk_v.at[q],
                                          [jnp.full((16,), r, jnp.int32)])
                    for j in range(D // 16):
                        sl = pl.ds(j * 16, 16)
                        rows_v[q, r, sl] = rows_v[q, r, sl] * av
                return u

            lax.fori_loop(0, _BE // 2, body_r, 0)

        def body_z(r, t):
            for j in range(D // 16):
                rows_v[0, r, pl.ds(j * 16, 16)] = zero16
            return t

        lax.fori_loop(0, _BE, body_z, 0)
        for j in range(_BE // 16):
            zrow_v[pl.ds(j * 16, 16)] = zero16
        for k in range(_NZC):
            base = s * _ROWS_PT + k * _BE
            pltpu.sync_copy(rows_v.at[0], sh_p.at[pl.ds(base, _BE)])
            pltpu.sync_copy(zrow_v, sh_d.at[pl.ds(base, _BE)])

        plsc.subcore_barrier()

        stage_start(0, 0)
        stage_start(1, 1)
        stage_start(2, 2)
        stage_wait(0, 0)
        gathers_start(0, 0)

        def body_g(g, t):
            for qq in range(4):
                b = g * 4 + qq
                slot = qq
                q = qq % 2

                @pl.when(b >= 1)
                def _():
                    scatter_wait((slot - 1) % 4, (q + 1) % 2)

                @pl.when(b + 1 < nblk)
                def _():
                    stage_wait(b + 1, (slot + 1) % 4)
                    gathers_start((slot + 1) % 4, (q + 1) % 2)

                @pl.when(b + 3 < nblk)
                def _():
                    stage_start(b + 3, (slot + 3) % 4)

                gathers_wait(slot, q)
                compute_block(b, slot, q)
                scatter_start(slot, q)
            return t

        lax.fori_loop(0, nblk // 4, body_g, 0)
        scatter_wait(3, 1)

        plsc.subcore_barrier()

        for k in range(_NZC):
            base = s * _ROWS_PT + k * _BE
            pltpu.sync_copy(sh_p.at[pl.ds(base, _BE)],
                            p_h.at[c].at[pl.ds(base, _BE)])

        @pl.when(s == 0)
        def _():
            pltpu.sync_copy(sh_d, d_h.at[c])

    return sc_fn


def _tc_pre_body(x_ref, w_ref, alr_ref, feat_ref, eler_ref, emax_ref):
    feat = jnp.dot(x_ref[...], w_ref[...], preferred_element_type=jnp.float32)
    feat_ref[...] = feat
    eler = jnp.dot(feat, alr_ref[...], preferred_element_type=jnp.float32)
    eler_ref[...] = eler
    emax_ref[...] = jnp.max(eler, axis=0, keepdims=True)


_tc_pre = pl.pallas_call(
    _tc_pre_body,
    out_shape=[
        jax.ShapeDtypeStruct((_N, _H), jnp.float32),
        jax.ShapeDtypeStruct((_N, 8), jnp.float32),
        jax.ShapeDtypeStruct((1, 8), jnp.float32),
    ],
)


def _make_tc_mid(has_res, Dn):
    def body(*refs):
        if has_res:
            (p0, p1, d0, d1, hprev, b, wn, alrn,
             h_ref, feat_ref, eler_ref, emax_ref) = refs
        else:
            (p0, p1, d0, d1, b, wn, alrn,
             h_ref, feat_ref, eler_ref, emax_ref) = refs
        num = p0[...] + p1[...]
        den = d0[...] + d1[...]
        agg = jnp.where(den > 0.0, num / den, 0.0)
        h = agg + b[...]
        if has_res:
            h = h + hprev[...]
        h = jnp.where(h > 0.0, h, jnp.exp(h) - 1.0)
        h_ref[...] = h
        feat = jnp.dot(h, wn[...], preferred_element_type=jnp.float32)
        feat_ref[...] = feat
        eler = jnp.dot(feat, alrn[...], preferred_element_type=jnp.float32)
        eler_ref[...] = eler
        emax_ref[...] = jnp.max(eler, axis=0, keepdims=True)

    return pl.pallas_call(
        body,
        out_shape=[
            jax.ShapeDtypeStruct((_N, _H), jnp.float32),
            jax.ShapeDtypeStruct((_N, Dn), jnp.float32),
            jax.ShapeDtypeStruct((_N, 8), jnp.float32),
            jax.ShapeDtypeStruct((1, 8), jnp.float32),
        ],
    )


_tc_mid0 = _make_tc_mid(False, _H)
_tc_mid1 = _make_tc_mid(True, _H)


def _tc_fin_body(p0, p1, d0, d1, hprev, wres, b, out_ref):
    num = p0[...] + p1[...]
    den = d0[...] + d1[...]
    agg = jnp.where(den > 0.0, num / den, 0.0)
    res = jnp.dot(hprev[...], wres[...], preferred_element_type=jnp.float32)
    out_ref[...] = agg + res + b[...]


_tc_fin = pl.pallas_call(
    _tc_fin_body,
    out_shape=jax.ShapeDtypeStruct((_N, _C), jnp.float32),
)


def _alr(al, ar):
    z = jnp.zeros_like(al)
    return jnp.stack([al, ar, z, z, z, z, z, z], axis=1)


def kernel(inputs, edge_index, W0, al0, ar0, b0, W1, al1, ar1, b1,
           W2, al2, ar2, b2, Wres2):
    sdp = jnp.pad(edge_index, ((0, 0), (0, _EP - _E)))

    def shift_vec(emax):
        m0 = emax[0, 0] + emax[0, 1]
        m = jnp.where(m0 >= 0.0, m0, _NEG * m0)
        return jnp.full((16,), m, jnp.float32)

    _sc_layer_h = _make_sc_layer(_H)
    zpad = jnp.zeros((_H, _H - _C), jnp.float32)
    W2p = jnp.concatenate([W2, zpad], axis=1)
    al2p = jnp.concatenate([al2, jnp.zeros((_H - _C,), jnp.float32)])
    ar2p = jnp.concatenate([ar2, jnp.zeros((_H - _C,), jnp.float32)])

    feat0, eler0, emax0 = _tc_pre(inputs, W0, _alr(al0, ar0))
    p, d = _sc_layer_h(feat0, sdp, shift_vec(emax0),
                       eler0[:, 0], eler0[:, 1])
    h1, feat1, eler1, emax1 = _tc_mid0(p[0, :_N], p[1, :_N], d[0, :_N, None],
                                       d[1, :_N, None], b0, W1,
                                       _alr(al1, ar1))

    p, d = _sc_layer_h(feat1, sdp, shift_vec(emax1),
                       eler1[:, 0], eler1[:, 1])
    h2, feat2, eler2, emax2 = _tc_mid1(p[0, :_N], p[1, :_N], d[0, :_N, None],
                                       d[1, :_N, None], h1, b1, W2p,
                                       _alr(al2p, ar2p))

    p, d = _sc_layer_h(feat2, sdp, shift_vec(emax2),
                       eler2[:, 0], eler2[:, 1])
    logits = _tc_fin(p[0, :_N, :_C], p[1, :_N, :_C], d[0, :_N, None],
                     d[1, :_N, None], h2, Wres2, b2)
    return logits


# speedup vs baseline: 1.5356x; 1.0000x over previous
import functools

import jax
import jax.numpy as jnp
from jax import lax
from jax.experimental import pallas as pl
from jax.experimental.pallas import tpu as pltpu
from jax.experimental.pallas import tpu_sc as plsc

_N = 10000
_E = 320000
_H = 128
_C = 64
_NP = 10240
_EP = 327680
_BE = 128
_EPT0 = 14336
_EPT1 = 6144
_NBLK0 = _EPT0 // _BE
_NBLK1 = _EPT1 // _BE
_ROWS_PT = _NP // 16
_NZC = _ROWS_PT // _BE
_NEG = 0.2


@functools.cache
def _make_sc_layer(D):
    mesh = plsc.VectorSubcoreMesh(core_axis_name="c", subcore_axis_name="s")

    @functools.partial(
        pl.kernel,
        out_type=[
            jax.ShapeDtypeStruct((2, _NP, D), jnp.float32),
            jax.ShapeDtypeStruct((2, _NP), jnp.float32),
        ],
        mesh=mesh,
        scratch_types=[
            pltpu.VMEM((2, _BE, D), jnp.float32),
            pltpu.VMEM((4, 2, _BE), jnp.int32),
            pltpu.VMEM((2, _BE), jnp.float32),
            pltpu.VMEM((2, _BE), jnp.float32),
            pltpu.VMEM((2, _BE), jnp.float32),
            pltpu.VMEM((_BE,), jnp.float32),
            pltpu.VMEM((16,), jnp.float32),
            pltpu.VMEM_SHARED((_NP, D), jnp.float32),
            pltpu.VMEM_SHARED((_NP,), jnp.float32),
            pltpu.VMEM_SHARED((_N,), jnp.float32),
            pltpu.VMEM_SHARED((_N,), jnp.float32),
            [pltpu.SemaphoreType.DMA] * 2,
            [pltpu.SemaphoreType.DMA] * 2,
            [pltpu.SemaphoreType.DMA] * 2,
            [pltpu.SemaphoreType.DMA] * 2,
            [pltpu.SemaphoreType.DMA] * 2,
            [pltpu.SemaphoreType.DMA] * 4,
        ],
        compiler_params=pltpu.CompilerParams(needs_layout_passes=False),
    )
    def sc_fn(feat_h, sd_h, mv_h, el_h, er_h, p_h, d_h,
              rows_v, sdblk_v, elblk_v, erblk_v, eeblk_v,
              zrow_v, mvec_v, sh_p, sh_d, sh_el, sh_er,
              gsem, elsem, ersem, spsem, sdsem, stsem):
        c = lax.axis_index("c")
        s = lax.axis_index("s")
        ebase = jnp.where(c == 0, s * _EPT0, 16 * _EPT0 + s * _EPT1)
        nblk = jnp.where(c == 0, _NBLK0, _NBLK1)

        pltpu.sync_copy(mv_h, mvec_v)
        Mv = mvec_v[...]

        @pl.when(s == 0)
        def _():
            pltpu.sync_copy(el_h, sh_el)
            pltpu.sync_copy(er_h, sh_er)

        iota16 = lax.broadcasted_iota(jnp.int32, (16,), 0)
        zero16 = jnp.zeros((16,), jnp.float32)

        def stage_start(b, slot):
            eb = ebase + b * _BE
            pltpu.async_copy(sd_h.at[:, pl.ds(eb, _BE)], sdblk_v.at[slot],
                             stsem[slot])

        def stage_wait(b, slot):
            eb = ebase + b * _BE
            pltpu.make_async_copy(sd_h.at[:, pl.ds(eb, _BE)],
                                  sdblk_v.at[slot], stsem[slot]).wait()

        def gathers_start(slot, q):
            pltpu.async_copy(sh_el.at[sdblk_v.at[slot, 0]], elblk_v.at[q],
                             elsem[q])
            pltpu.async_copy(sh_er.at[sdblk_v.at[slot, 1]], erblk_v.at[q],
                             ersem[q])
            hb = _BE // 2
            pltpu.async_copy(feat_h.at[sdblk_v.at[slot, 0].at[pl.ds(0, hb)]],
                             rows_v.at[q].at[pl.ds(0, hb)], gsem[q])
            pltpu.async_copy(feat_h.at[sdblk_v.at[slot, 0].at[pl.ds(hb, hb)]],
                             rows_v.at[q].at[pl.ds(hb, hb)], gsem[q])

        def gathers_wait(slot, q):
            pltpu.make_async_copy(sh_el.at[sdblk_v.at[slot, 0]],
                                  elblk_v.at[q], elsem[q]).wait()
            pltpu.make_async_copy(sh_er.at[sdblk_v.at[slot, 1]],
                                  erblk_v.at[q], ersem[q]).wait()
            hb = _BE // 2
            pltpu.make_async_copy(feat_h.at[sdblk_v.at[slot, 0].at[pl.ds(0, hb)]],
                                  rows_v.at[q].at[pl.ds(0, hb)], gsem[q]).wait()
            pltpu.make_async_copy(feat_h.at[sdblk_v.at[slot, 0].at[pl.ds(hb, hb)]],
                                  rows_v.at[q].at[pl.ds(hb, hb)], gsem[q]).wait()

        def scatter_start(slot, q):
            pltpu.async_copy(eeblk_v.at[q], sh_d.at[sdblk_v.at[slot, 1]],
                             sdsem[q], add=True)
            pltpu.async_copy(rows_v.at[q], sh_p.at[sdblk_v.at[slot, 1]],
                             spsem[q], add=True)

        def scatter_wait(slot, q):
            pltpu.make_async_copy(eeblk_v.at[q], sh_d.at[sdblk_v.at[slot, 1]],
                                  sdsem[q]).wait()
            pltpu.make_async_copy(rows_v.at[q], sh_p.at[sdblk_v.at[slot, 1]],
                                  spsem[q]).wait()

        def compute_block(b, slot, q):
            eb = ebase + b * _BE
            for j in range(_BE // 16):
                sl = pl.ds(j * 16, 16)
                e = elblk_v[q, sl] + erblk_v[q, sl]
                e = jnp.where(e >= 0.0, e, _NEG * e)
                gid = eb + j * 16 + iota16
                e = jnp.where(gid < _E, e, -1e30)
                eeblk_v[q, sl] = jnp.exp(e - Mv)

            def body_r(i, u):
                for rr in range(2):
                    r = i * 2 + rr
                    av = plsc.load_gather(eeblk_v.at[q],
                                          [jnp.full((16,), r, jnp.int32)])
                    for j in range(D // 16):
                        sl = pl.ds(j * 16, 16)
                        rows_v[q, r, sl] = rows_v[q, r, sl] * av
                return u

            lax.fori_loop(0, _BE // 2, body_r, 0)

        def body_z(r, t):
            for j in range(D // 16):
                rows_v[0, r, pl.ds(j * 16, 16)] = zero16
            return t

        lax.fori_loop(0, _BE, body_z, 0)
        for j in range(_BE // 16):
            zrow_v[pl.ds(j * 16, 16)] = zero16
        for k in range(_NZC):
            base = s * _ROWS_PT + k * _BE
            pltpu.sync_copy(rows_v.at[0], sh_p.at[pl.ds(base, _BE)])
            pltpu.sync_copy(zrow_v, sh_d.at[pl.ds(base, _BE)])

        plsc.subcore_barrier()

        stage_start(0, 0)
        stage_start(1, 1)
        stage_start(2, 2)
        stage_wait(0, 0)
        gathers_start(0, 0)

        def body_g(g, t):
            for qq in range(4):
                b = g * 4 + qq
                slot = qq
                q = qq % 2

                @pl.when(b >= 1)
                def _():
                    scatter_wait((slot - 1) % 4, (q + 1) % 2)

                @pl.when(b + 1 < nblk)
                def _():
                    stage_wait(b + 1, (slot + 1) % 4)
                    gathers_start((slot + 1) % 4, (q + 1) % 2)

                @pl.when(b + 3 < nblk)
                def _():
                    stage_start(b + 3, (slot + 3) % 4)

                gathers_wait(slot, q)
                compute_block(b, slot, q)
                scatter_start(slot, q)
            return t

        lax.fori_loop(0, nblk // 4, body_g, 0)
        scatter_wait(3, 1)

        plsc.subcore_barrier()

        for k in range(_NZC):
            base = s * _ROWS_PT + k * _BE
            pltpu.sync_copy(sh_p.at[pl.ds(base, _BE)],
                            p_h.at[c].at[pl.ds(base, _BE)])

        @pl.when(s == 0)
        def _():
            pltpu.sync_copy(sh_d, d_h.at[c])

    return sc_fn


def _tc_pre_body(x_ref, w_ref, alr_ref, feat_ref, eler_ref, emax_ref):
    feat = jnp.dot(x_ref[...], w_ref[...], preferred_element_type=jnp.float32)
    feat_ref[...] = feat
    eler = jnp.dot(feat, alr_ref[...], preferred_element_type=jnp.float32)
    eler_ref[...] = eler
    emax_ref[...] = jnp.max(eler, axis=0, keepdims=True)


_tc_pre = pl.pallas_call(
    _tc_pre_body,
    out_shape=[
        jax.ShapeDtypeStruct((_N, _H), jnp.float32),
        jax.ShapeDtypeStruct((_N, 8), jnp.float32),
        jax.ShapeDtypeStruct((1, 8), jnp.float32),
    ],
)


def _make_tc_mid(has_res, Dn):
    def body(*refs):
        if has_res:
            (p0, p1, d0, d1, hprev, b, wn, alrn,
             h_ref, feat_ref, eler_ref, emax_ref) = refs
        else:
            (p0, p1, d0, d1, b, wn, alrn,
             h_ref, feat_ref, eler_ref, emax_ref) = refs
        num = p0[...] + p1[...]
        den = d0[...] + d1[...]
        agg = jnp.where(den > 0.0, num / den, 0.0)
        h = agg + b[...]
        if has_res:
            h = h + hprev[...]
        h = jnp.where(h > 0.0, h, jnp.exp(h) - 1.0)
        h_ref[...] = h
        feat = jnp.dot(h, wn[...], preferred_element_type=jnp.float32)
        feat_ref[...] = feat
        eler = jnp.dot(feat, alrn[...], preferred_element_type=jnp.float32)
        eler_ref[...] = eler
        emax_ref[...] = jnp.max(eler, axis=0, keepdims=True)

    return pl.pallas_call(
        body,
        out_shape=[
            jax.ShapeDtypeStruct((_N, _H), jnp.float32),
            jax.ShapeDtypeStruct((_N, Dn), jnp.float32),
            jax.ShapeDtypeStruct((_N, 8), jnp.float32),
            jax.ShapeDtypeStruct((1, 8), jnp.float32),
        ],
    )


_tc_mid0 = _make_tc_mid(False, _H)
_tc_mid1 = _make_tc_mid(True, _H)


def _tc_fin_body(p0, p1, d0, d1, hprev, wres, b, out_ref):
    num = p0[...] + p1[...]
    den = d0[...] + d1[...]
    agg = jnp.where(den > 0.0, num / den, 0.0)
    res = jnp.dot(hprev[...], wres[...], preferred_element_type=jnp.float32)
    out_ref[...] = agg + res + b[...]


_tc_fin = pl.pallas_call(
    _tc_fin_body,
    out_shape=jax.ShapeDtypeStruct((_N, _C), jnp.float32),
)


def _alr(al, ar):
    z = jnp.zeros_like(al)
    return jnp.stack([al, ar, z, z, z, z, z, z], axis=1)


def kernel(inputs, edge_index, W0, al0, ar0, b0, W1, al1, ar1, b1,
           W2, al2, ar2, b2, Wres2):
    sdp = jnp.pad(edge_index, ((0, 0), (0, _EP - _E)))

    def shift_vec(emax):
        m0 = emax[0, 0] + emax[0, 1]
        m = jnp.where(m0 >= 0.0, m0, _NEG * m0)
        return jnp.full((16,), m, jnp.float32)

    _sc_layer_h = _make_sc_layer(_H)
    zpad = jnp.zeros((_H, _H - _C), jnp.float32)
    W2p = jnp.concatenate([W2, zpad], axis=1)
    al2p = jnp.concatenate([al2, jnp.zeros((_H - _C,), jnp.float32)])
    ar2p = jnp.concatenate([ar2, jnp.zeros((_H - _C,), jnp.float32)])

    feat0, eler0, emax0 = _tc_pre(inputs, W0, _alr(al0, ar0))
    p, d = _sc_layer_h(feat0, sdp, shift_vec(emax0),
                       eler0[:, 0], eler0[:, 1])
    h1, feat1, eler1, emax1 = _tc_mid0(p[0, :_N], p[1, :_N], d[0, :_N, None],
                                       d[1, :_N, None], b0, W1,
                                       _alr(al1, ar1))

    p, d = _sc_layer_h(feat1, sdp, shift_vec(emax1),
                       eler1[:, 0], eler1[:, 1])
    h2, feat2, eler2, emax2 = _tc_mid1(p[0, :_N], p[1, :_N], d[0, :_N, None],
                                       d[1, :_N, None], h1, b1, W2p,
                                       _alr(al2p, ar2p))

    p, d = _sc_layer_h(feat2, sdp, shift_vec(emax2),
                       eler2[:, 0], eler2[:, 1])
    logits = _tc_fin(p[0, :_N, :_C], p[1, :_N, :_C], d[0, :_N, None],
                     d[1, :_N, None], h2, Wres2, b2)
    return logits


# speedup vs baseline: 1.5372x; 1.0011x over previous
import functools

import jax
import jax.numpy as jnp
from jax import lax
from jax.experimental import pallas as pl
from jax.experimental.pallas import tpu as pltpu
from jax.experimental.pallas import tpu_sc as plsc

_N = 10000
_E = 320000
_H = 128
_C = 64
_NP = 10240
_EP = 327680
_BE = 128
_EPT0 = 14336
_EPT1 = 6144
_NBLK0 = _EPT0 // _BE
_NBLK1 = _EPT1 // _BE
_ROWS_PT = _NP // 16
_NZC = _ROWS_PT // _BE
_NEG = 0.2


@functools.cache
def _make_sc_layer(D):
    mesh = plsc.VectorSubcoreMesh(core_axis_name="c", subcore_axis_name="s")

    @functools.partial(
        pl.kernel,
        out_type=[
            jax.ShapeDtypeStruct((2, _NP, D), jnp.float32),
            jax.ShapeDtypeStruct((2, _NP), jnp.float32),
        ],
        mesh=mesh,
        scratch_types=[
            pltpu.VMEM((2, _BE, D), jnp.float32),
            pltpu.VMEM((4, 2, _BE), jnp.int32),
            pltpu.VMEM((2, _BE), jnp.float32),
            pltpu.VMEM((2, _BE), jnp.float32),
            pltpu.VMEM((2, _BE), jnp.float32),
            pltpu.VMEM((_BE,), jnp.float32),
            pltpu.VMEM((16,), jnp.float32),
            pltpu.VMEM_SHARED((_NP, D), jnp.float32),
            pltpu.VMEM_SHARED((_NP,), jnp.float32),
            pltpu.VMEM_SHARED((_N,), jnp.float32),
            pltpu.VMEM_SHARED((_N,), jnp.float32),
            [pltpu.SemaphoreType.DMA] * 2,
            [pltpu.SemaphoreType.DMA] * 2,
            [pltpu.SemaphoreType.DMA] * 2,
            [pltpu.SemaphoreType.DMA] * 2,
            [pltpu.SemaphoreType.DMA] * 2,
            [pltpu.SemaphoreType.DMA] * 4,
        ],
        compiler_params=pltpu.CompilerParams(needs_layout_passes=False),
    )
    def sc_fn(feat_h, sd_h, mv_h, el_h, er_h, p_h, d_h,
              rows_v, sdblk_v, elblk_v, erblk_v, eeblk_v,
              zrow_v, mvec_v, sh_p, sh_d, sh_el, sh_er,
              gsem, elsem, ersem, spsem, sdsem, stsem):
        c = lax.axis_index("c")
        s = lax.axis_index("s")
        ebase = jnp.where(c == 0, s * _EPT0, 16 * _EPT0 + s * _EPT1)
        nblk = jnp.where(c == 0, _NBLK0, _NBLK1)

        pltpu.sync_copy(mv_h, mvec_v)
        Mv = mvec_v[...]

        @pl.when(s == 0)
        def _():
            pltpu.sync_copy(el_h, sh_el)
            pltpu.sync_copy(er_h, sh_er)

        iota16 = lax.broadcasted_iota(jnp.int32, (16,), 0)
        zero16 = jnp.zeros((16,), jnp.float32)

        def stage_start(b, slot):
            eb = ebase + b * _BE
            pltpu.async_copy(sd_h.at[:, pl.ds(eb, _BE)], sdblk_v.at[slot],
                             stsem[slot])

        def stage_wait(b, slot):
            eb = ebase + b * _BE
            pltpu.make_async_copy(sd_h.at[:, pl.ds(eb, _BE)],
                                  sdblk_v.at[slot], stsem[slot]).wait()

        def gathers_start(slot, q):
            pltpu.async_copy(sh_el.at[sdblk_v.at[slot, 0]], elblk_v.at[q],
                             elsem[q])
            pltpu.async_copy(sh_er.at[sdblk_v.at[slot, 1]], erblk_v.at[q],
                             ersem[q])
            hb = _BE // 2
            pltpu.async_copy(feat_h.at[sdblk_v.at[slot, 0].at[pl.ds(0, hb)]],
                             rows_v.at[q].at[pl.ds(0, hb)], gsem[q])
            pltpu.async_copy(feat_h.at[sdblk_v.at[slot, 0].at[pl.ds(hb, hb)]],
                             rows_v.at[q].at[pl.ds(hb, hb)], gsem[q])

        def gathers_wait(slot, q):
            pltpu.make_async_copy(sh_el.at[sdblk_v.at[slot, 0]],
                                  elblk_v.at[q], elsem[q]).wait()
            pltpu.make_async_copy(sh_er.at[sdblk_v.at[slot, 1]],
                                  erblk_v.at[q], ersem[q]).wait()
            hb = _BE // 2
            pltpu.make_async_copy(feat_h.at[sdblk_v.at[slot, 0].at[pl.ds(0, hb)]],
                                  rows_v.at[q].at[pl.ds(0, hb)], gsem[q]).wait()
            pltpu.make_async_copy(feat_h.at[sdblk_v.at[slot, 0].at[pl.ds(hb, hb)]],
                                  rows_v.at[q].at[pl.ds(hb, hb)], gsem[q]).wait()

        def scatter_start(slot, q):
            pltpu.async_copy(eeblk_v.at[q], sh_d.at[sdblk_v.at[slot, 1]],
                             sdsem[q], add=True)
            pltpu.async_copy(rows_v.at[q], sh_p.at[sdblk_v.at[slot, 1]],
                             spsem[q], add=True)

        def scatter_wait(slot, q):
            pltpu.make_async_copy(eeblk_v.at[q], sh_d.at[sdblk_v.at[slot, 1]],
                                  sdsem[q]).wait()
            pltpu.make_async_copy(rows_v.at[q], sh_p.at[sdblk_v.at[slot, 1]],
                                  spsem[q]).wait()

        def compute_block(b, slot, q):
            eb = ebase + b * _BE
            for j in range(_BE // 16):
                sl = pl.ds(j * 16, 16)
                e = elblk_v[q, sl] + erblk_v[q, sl]
                e = jnp.where(e >= 0.0, e, _NEG * e)
                gid = eb + j * 16 + iota16
                e = jnp.where(gid < _E, e, -1e30)
                eeblk_v[q, sl] = jnp.exp(e - Mv)

            def body_r(i, u):
                for rr in range(2):
                    r = i * 2 + rr
                    av = plsc.load_gather(eeblk_v.at[q],
                                          [jnp.full((16,), r, jnp.int32)])
                    for j in range(D // 16):
                        sl = pl.ds(j * 16, 16)
                        rows_v[q, r, sl] = rows_v[q, r, sl] * av
                return u

            lax.fori_loop(0, _BE // 2, body_r, 0)

        def body_z(r, t):
            for j in range(D // 16):
                rows_v[0, r, pl.ds(j * 16, 16)] = zero16
            return t

        lax.fori_loop(0, _BE, body_z, 0)
        for j in range(_BE // 16):
            zrow_v[pl.ds(j * 16, 16)] = zero16
        for k in range(_NZC):
            base = s * _ROWS_PT + k * _BE
            pltpu.async_copy(rows_v.at[0], sh_p.at[pl.ds(base, _BE)],
                             spsem[0])
            pltpu.async_copy(zrow_v, sh_d.at[pl.ds(base, _BE)], sdsem[0])
        for k in range(_NZC):
            base = s * _ROWS_PT + k * _BE
            pltpu.make_async_copy(rows_v.at[0], sh_p.at[pl.ds(base, _BE)],
                                  spsem[0]).wait()
            pltpu.make_async_copy(zrow_v, sh_d.at[pl.ds(base, _BE)],
                                  sdsem[0]).wait()

        plsc.subcore_barrier()

        stage_start(0, 0)
        stage_start(1, 1)
        stage_start(2, 2)
        stage_wait(0, 0)
        gathers_start(0, 0)

        def body_g(g, t):
            for qq in range(4):
                b = g * 4 + qq
                slot = qq
                q = qq % 2

                @pl.when(b >= 1)
                def _():
                    scatter_wait((slot - 1) % 4, (q + 1) % 2)

                @pl.when(b + 1 < nblk)
                def _():
                    stage_wait(b + 1, (slot + 1) % 4)
                    gathers_start((slot + 1) % 4, (q + 1) % 2)

                @pl.when(b + 3 < nblk)
                def _():
                    stage_start(b + 3, (slot + 3) % 4)

                gathers_wait(slot, q)
                compute_block(b, slot, q)
                scatter_start(slot, q)
            return t

        lax.fori_loop(0, nblk // 4, body_g, 0)
        scatter_wait(3, 1)

        plsc.subcore_barrier()

        for k in range(_NZC):
            base = s * _ROWS_PT + k * _BE
            pltpu.async_copy(sh_p.at[pl.ds(base, _BE)],
                             p_h.at[c].at[pl.ds(base, _BE)], gsem[0])

        @pl.when(s == 0)
        def _():
            pltpu.async_copy(sh_d, d_h.at[c], gsem[1])

        for k in range(_NZC):
            base = s * _ROWS_PT + k * _BE
            pltpu.make_async_copy(sh_p.at[pl.ds(base, _BE)],
                                  p_h.at[c].at[pl.ds(base, _BE)],
                                  gsem[0]).wait()

        @pl.when(s == 0)
        def _():
            pltpu.make_async_copy(sh_d, d_h.at[c], gsem[1]).wait()

    return sc_fn


def _tc_pre_body(x_ref, w_ref, alr_ref, feat_ref, eler_ref, emax_ref):
    feat = jnp.dot(x_ref[...], w_ref[...], preferred_element_type=jnp.float32)
    feat_ref[...] = feat
    eler = jnp.dot(feat, alr_ref[...], preferred_element_type=jnp.float32)
    eler_ref[...] = eler
    emax_ref[...] = jnp.max(eler, axis=0, keepdims=True)


_tc_pre = pl.pallas_call(
    _tc_pre_body,
    out_shape=[
        jax.ShapeDtypeStruct((_N, _H), jnp.float32),
        jax.ShapeDtypeStruct((_N, 8), jnp.float32),
        jax.ShapeDtypeStruct((1, 8), jnp.float32),
    ],
)


def _make_tc_mid(has_res, Dn):
    def body(*refs):
        if has_res:
            (p0, p1, d0, d1, hprev, b, wn, alrn,
             h_ref, feat_ref, eler_ref, emax_ref) = refs
        else:
            (p0, p1, d0, d1, b, wn, alrn,
             h_ref, feat_ref, eler_ref, emax_ref) = refs
        num = p0[...] + p1[...]
        den = d0[...] + d1[...]
        agg = jnp.where(den > 0.0, num / den, 0.0)
        h = agg + b[...]
        if has_res:
            h = h + hprev[...]
        h = jnp.where(h > 0.0, h, jnp.exp(h) - 1.0)
        h_ref[...] = h
        feat = jnp.dot(h, wn[...], preferred_element_type=jnp.float32)
        feat_ref[...] = feat
        eler = jnp.dot(feat, alrn[...], preferred_element_type=jnp.float32)
        eler_ref[...] = eler
        emax_ref[...] = jnp.max(eler, axis=0, keepdims=True)

    return pl.pallas_call(
        body,
        out_shape=[
            jax.ShapeDtypeStruct((_N, _H), jnp.float32),
            jax.ShapeDtypeStruct((_N, Dn), jnp.float32),
            jax.ShapeDtypeStruct((_N, 8), jnp.float32),
            jax.ShapeDtypeStruct((1, 8), jnp.float32),
        ],
    )


_tc_mid0 = _make_tc_mid(False, _H)
_tc_mid1 = _make_tc_mid(True, _H)


def _tc_fin_body(p0, p1, d0, d1, hprev, wres, b, out_ref):
    num = p0[...] + p1[...]
    den = d0[...] + d1[...]
    agg = jnp.where(den > 0.0, num / den, 0.0)
    res = jnp.dot(hprev[...], wres[...], preferred_element_type=jnp.float32)
    out_ref[...] = agg + res + b[...]


_tc_fin = pl.pallas_call(
    _tc_fin_body,
    out_shape=jax.ShapeDtypeStruct((_N, _C), jnp.float32),
)


def _alr(al, ar):
    z = jnp.zeros_like(al)
    return jnp.stack([al, ar, z, z, z, z, z, z], axis=1)


def kernel(inputs, edge_index, W0, al0, ar0, b0, W1, al1, ar1, b1,
           W2, al2, ar2, b2, Wres2):
    sdp = jnp.pad(edge_index, ((0, 0), (0, _EP - _E)))

    def shift_vec(emax):
        m0 = emax[0, 0] + emax[0, 1]
        m = jnp.where(m0 >= 0.0, m0, _NEG * m0)
        return jnp.full((16,), m, jnp.float32)

    _sc_layer_h = _make_sc_layer(_H)
    zpad = jnp.zeros((_H, _H - _C), jnp.float32)
    W2p = jnp.concatenate([W2, zpad], axis=1)
    al2p = jnp.concatenate([al2, jnp.zeros((_H - _C,), jnp.float32)])
    ar2p = jnp.concatenate([ar2, jnp.zeros((_H - _C,), jnp.float32)])

    feat0, eler0, emax0 = _tc_pre(inputs, W0, _alr(al0, ar0))
    p, d = _sc_layer_h(feat0, sdp, shift_vec(emax0),
                       eler0[:, 0], eler0[:, 1])
    h1, feat1, eler1, emax1 = _tc_mid0(p[0, :_N], p[1, :_N], d[0, :_N, None],
                                       d[1, :_N, None], b0, W1,
                                       _alr(al1, ar1))

    p, d = _sc_layer_h(feat1, sdp, shift_vec(emax1),
                       eler1[:, 0], eler1[:, 1])
    h2, feat2, eler2, emax2 = _tc_mid1(p[0, :_N], p[1, :_N], d[0, :_N, None],
                                       d[1, :_N, None], h1, b1, W2p,
                                       _alr(al2p, ar2p))

    p, d = _sc_layer_h(feat2, sdp, shift_vec(emax2),
                       eler2[:, 0], eler2[:, 1])
    logits = _tc_fin(p[0, :_N, :_C], p[1, :_N, :_C], d[0, :_N, None],
                     d[1, :_N, None], h2, Wres2, b2)
    return logits
